# Initial kernel scaffold; baseline (speedup 1.0000x reference)
import jax
import jax.numpy as jnp
from jax.experimental import pallas as pl


def kernel(text, offsets, emb_weight, fc_weight, fc_bias):
    raise NotImplementedError("write your pallas kernel here")


# speedup vs baseline: 30.1423x; 30.1423x over previous
import functools

import jax
import jax.numpy as jnp
from jax import lax
from jax.experimental import pallas as pl
from jax.experimental.pallas import tpu as pltpu
from jax.experimental.pallas import tpu_sc as plsc

VOCAB = 1000000
EMBED = 64
NUM_CLASS = 16
BATCH = 4096
BAG = 50

NC = 2
NS = 16
LANES = 16
NW = NC * NS
VPR = EMBED // LANES

BAGS_PER_W = BATCH // NW
CHUNK_BAGS = 2
CHUNK_TOK = CHUNK_BAGS * BAG
NCHUNK = BAGS_PER_W // CHUNK_BAGS


def _sc_bag_sums_body(text_hbm, emb_hbm, out_hbm, idx_v, buf_v, acc_v, sem):
    wid = lax.axis_index("s") * NC + lax.axis_index("c")
    row0 = wid * NCHUNK
    pltpu.sync_copy(text_hbm.at[pl.ds(row0, NCHUNK)], idx_v)

    def chunk(g, carry):
        cp = pltpu.async_copy(emb_hbm.at[idx_v.at[g]], buf_v, sem)
        cp.wait()
        for h in range(CHUNK_BAGS):
            accs = [jnp.zeros((LANES,), jnp.float32) for _ in range(VPR)]
            for t in range(BAG):
                r = h * BAG + t
                for j in range(VPR):
                    accs[j] = accs[j] + buf_v[r, pl.ds(j * LANES, LANES)]
            b = g * CHUNK_BAGS + h
            for j in range(VPR):
                acc_v[b, pl.ds(j * LANES, LANES)] = accs[j]
        return carry

    lax.fori_loop(0, NCHUNK, chunk, 0)
    pltpu.sync_copy(acc_v, out_hbm.at[pl.ds(wid * BAGS_PER_W, BAGS_PER_W)])


_sc_bag_sums = pl.kernel(
    _sc_bag_sums_body,
    out_type=jax.ShapeDtypeStruct((BATCH, EMBED), jnp.float32),
    mesh=plsc.VectorSubcoreMesh(
        core_axis_name="c", subcore_axis_name="s",
        num_cores=NC, num_subcores=NS),
    scratch_types=[
        pltpu.VMEM((NCHUNK, CHUNK_TOK), jnp.int32),
        pltpu.VMEM((CHUNK_TOK, EMBED), jnp.float32),
        pltpu.VMEM((BAGS_PER_W, EMBED), jnp.float32),
        pltpu.SemaphoreType.DMA,
    ],
    compiler_params=pltpu.CompilerParams(use_tc_tiling_on_sc=False),
)


def _tc_linear_body(x_ref, w_ref, b_ref, o_ref):
    x = x_ref[...]
    w = w_ref[...]
    acc = lax.dot_general(x, w, (((1,), (1,)), ((), ())),
                          preferred_element_type=jnp.float32)
    o_ref[...] = acc * (1.0 / BAG) + b_ref[...]


_tc_linear = pl.pallas_call(
    _tc_linear_body,
    out_shape=jax.ShapeDtypeStruct((BATCH, NUM_CLASS), jnp.float32),
)


def kernel(text, offsets, emb_weight, fc_weight, fc_bias):
    del offsets
    text2d = text.reshape(BATCH * BAG // CHUNK_TOK, CHUNK_TOK).astype(jnp.int32)
    sums = _sc_bag_sums(text2d, emb_weight)
    return _tc_linear(sums, fc_weight, fc_bias.reshape(1, NUM_CLASS))


# speedup vs baseline: 31.5513x; 1.0467x over previous
import functools

import jax
import jax.numpy as jnp
from jax import lax
from jax.experimental import pallas as pl
from jax.experimental.pallas import tpu as pltpu
from jax.experimental.pallas import tpu_sc as plsc

VOCAB = 1000000
EMBED = 64
NUM_CLASS = 16
BATCH = 4096
BAG = 50

NC = 2
NS = 16
LANES = 16
NW = NC * NS
VPR = EMBED // LANES

BAGS_PER_W = BATCH // NW
CHUNK_BAGS = 2
CHUNK_TOK = CHUNK_BAGS * BAG
NCHUNK = BAGS_PER_W // CHUNK_BAGS


NBUF = 4


def _sc_bag_sums_body(text_hbm, emb_hbm, out_hbm, idx_v, bufs, acc_v, sems):
    wid = lax.axis_index("s") * NC + lax.axis_index("c")
    row0 = wid * NCHUNK
    pltpu.sync_copy(text_hbm.at[pl.ds(row0, NCHUNK)], idx_v)

    def fire(g, b):
        pltpu.async_copy(emb_hbm.at[idx_v.at[g]], bufs[b], sems[b])

    def wait(g, b):
        pltpu.make_async_copy(emb_hbm.at[idx_v.at[g]], bufs[b], sems[b]).wait()

    def process(g, b):
        buf_v = bufs[b]
        for h in range(CHUNK_BAGS):
            accs = [jnp.zeros((LANES,), jnp.float32) for _ in range(VPR)]
            for t in range(BAG):
                r = h * BAG + t
                for j in range(VPR):
                    accs[j] = accs[j] + buf_v[r, pl.ds(j * LANES, LANES)]
            bag = g * CHUNK_BAGS + h
            for j in range(VPR):
                acc_v[bag, pl.ds(j * LANES, LANES)] = accs[j]

    for b in range(NBUF):
        fire(b, b)

    def group(i, carry):
        for b in range(NBUF):
            g = i * NBUF + b
            wait(g, b)
            process(g, b)
            nxt = g + NBUF

            @pl.when(nxt < NCHUNK)
            def _():
                fire(nxt, b)
        return carry

    lax.fori_loop(0, NCHUNK // NBUF, group, 0)
    pltpu.sync_copy(acc_v, out_hbm.at[pl.ds(wid * BAGS_PER_W, BAGS_PER_W)])


_sc_bag_sums = pl.kernel(
    _sc_bag_sums_body,
    out_type=jax.ShapeDtypeStruct((BATCH, EMBED), jnp.float32),
    mesh=plsc.VectorSubcoreMesh(
        core_axis_name="c", subcore_axis_name="s",
        num_cores=NC, num_subcores=NS),
    scratch_types=[
        pltpu.VMEM((NCHUNK, CHUNK_TOK), jnp.int32),
        [pltpu.VMEM((CHUNK_TOK, EMBED), jnp.float32) for _ in range(NBUF)],
        pltpu.VMEM((BAGS_PER_W, EMBED), jnp.float32),
        [pltpu.SemaphoreType.DMA for _ in range(NBUF)],
    ],
    compiler_params=pltpu.CompilerParams(use_tc_tiling_on_sc=False),
)


def _tc_linear_body(x_ref, w_ref, b_ref, o_ref):
    x = x_ref[...]
    w = w_ref[...]
    acc = lax.dot_general(x, w, (((1,), (1,)), ((), ())),
                          preferred_element_type=jnp.float32)
    o_ref[...] = acc * (1.0 / BAG) + b_ref[...]


_tc_linear = pl.pallas_call(
    _tc_linear_body,
    out_shape=jax.ShapeDtypeStruct((BATCH, NUM_CLASS), jnp.float32),
)


def kernel(text, offsets, emb_weight, fc_weight, fc_bias):
    del offsets
    text2d = text.reshape(BATCH * BAG // CHUNK_TOK, CHUNK_TOK).astype(jnp.int32)
    sums = _sc_bag_sums(text2d, emb_weight)
    return _tc_linear(sums, fc_weight, fc_bias.reshape(1, NUM_CLASS))


# speedup vs baseline: 35.8984x; 1.1378x over previous
import functools

import jax
import jax.numpy as jnp
from jax import lax
from jax.experimental import pallas as pl
from jax.experimental.pallas import tpu as pltpu
from jax.experimental.pallas import tpu_sc as plsc

VOCAB = 1000000
EMBED = 64
NUM_CLASS = 16
BATCH = 4096
BAG = 50

NC = 2
NS = 16
LANES = 16
NW = NC * NS

BAGS_PER_W = BATCH // NW
CHUNK_BAGS = 2
CHUNK_TOK = CHUNK_BAGS * BAG
NCHUNK = BAGS_PER_W // CHUNK_BAGS
NBUF = 4

PROJ_BLK = 16384


def _tc_project_body(et_ref, w_ref, o_ref):
    o_ref[...] = lax.dot_general(
        et_ref[...], w_ref[...], (((0,), (1,)), ((), ())),
        preferred_element_type=jnp.float32)


_tc_project = pl.pallas_call(
    _tc_project_body,
    grid=(pl.cdiv(VOCAB, PROJ_BLK),),
    in_specs=[
        pl.BlockSpec((EMBED, PROJ_BLK), lambda i: (0, i)),
        pl.BlockSpec((NUM_CLASS, EMBED), lambda i: (0, 0)),
    ],
    out_specs=pl.BlockSpec((PROJ_BLK, NUM_CLASS), lambda i: (i, 0)),
    out_shape=jax.ShapeDtypeStruct((VOCAB, NUM_CLASS), jnp.float32),
)


def _sc_bag_mean_body(text_hbm, p_hbm, bias_hbm, out_hbm,
                      idx_v, bufs, acc_v, bias_v, sems):
    wid = lax.axis_index("s") * NC + lax.axis_index("c")
    row0 = wid * NCHUNK
    pltpu.sync_copy(text_hbm.at[pl.ds(row0, NCHUNK)], idx_v)
    pltpu.sync_copy(bias_hbm, bias_v)
    bias = bias_v[...]

    def fire(g, b):
        pltpu.async_copy(p_hbm.at[idx_v.at[g]], bufs[b], sems[b])

    def wait(g, b):
        pltpu.make_async_copy(p_hbm.at[idx_v.at[g]], bufs[b], sems[b]).wait()

    def process(g, b):
        buf_v = bufs[b]
        for h in range(CHUNK_BAGS):
            acc = jnp.zeros((NUM_CLASS,), jnp.float32)
            for t in range(BAG):
                acc = acc + buf_v[h * BAG + t, :]
            acc_v[g * CHUNK_BAGS + h, :] = acc * (1.0 / BAG) + bias

    for b in range(NBUF):
        fire(b, b)

    def group(i, carry):
        for b in range(NBUF):
            g = i * NBUF + b
            wait(g, b)
            process(g, b)
            nxt = g + NBUF

            @pl.when(nxt < NCHUNK)
            def _():
                fire(nxt, b)

        return carry

    lax.fori_loop(0, NCHUNK // NBUF, group, 0)
    pltpu.sync_copy(acc_v, out_hbm.at[pl.ds(wid * BAGS_PER_W, BAGS_PER_W)])


_sc_bag_mean = pl.kernel(
    _sc_bag_mean_body,
    out_type=jax.ShapeDtypeStruct((BATCH, NUM_CLASS), jnp.float32),
    mesh=plsc.VectorSubcoreMesh(
        core_axis_name="c", subcore_axis_name="s",
        num_cores=NC, num_subcores=NS),
    scratch_types=[
        pltpu.VMEM((NCHUNK, CHUNK_TOK), jnp.int32),
        [pltpu.VMEM((CHUNK_TOK, NUM_CLASS), jnp.float32) for _ in range(NBUF)],
        pltpu.VMEM((BAGS_PER_W, NUM_CLASS), jnp.float32),
        pltpu.VMEM((NUM_CLASS,), jnp.float32),
        [pltpu.SemaphoreType.DMA for _ in range(NBUF)],
    ],
    compiler_params=pltpu.CompilerParams(use_tc_tiling_on_sc=False),
)


def kernel(text, offsets, emb_weight, fc_weight, fc_bias):
    del offsets
    proj = _tc_project(emb_weight.T, fc_weight)
    text2d = text.reshape(BATCH * BAG // CHUNK_TOK, CHUNK_TOK).astype(jnp.int32)
    return _sc_bag_mean(text2d, proj, fc_bias)


# speedup vs baseline: 35.9285x; 1.0008x over previous
import functools

import jax
import jax.numpy as jnp
from jax import lax
from jax.experimental import pallas as pl
from jax.experimental.pallas import tpu as pltpu
from jax.experimental.pallas import tpu_sc as plsc

VOCAB = 1000000
EMBED = 64
NUM_CLASS = 16
BATCH = 4096
BAG = 50

NC = 2
NS = 16
LANES = 16
NW = NC * NS

BAGS_PER_W = BATCH // NW
CHUNK_BAGS = 2
CHUNK_TOK = CHUNK_BAGS * BAG
NCHUNK = BAGS_PER_W // CHUNK_BAGS
NBUF = 4

PROJ_BLK = 16384


def _tc_project_body(et_ref, w_ref, o_ref):
    q = lax.dot_general(
        w_ref[...], et_ref[...], (((1,), (0,)), ((), ())),
        preferred_element_type=jnp.float32)
    o_ref[...] = q.T


_tc_project = pl.pallas_call(
    _tc_project_body,
    grid=(pl.cdiv(VOCAB, PROJ_BLK),),
    in_specs=[
        pl.BlockSpec((EMBED, PROJ_BLK), lambda i: (0, i)),
        pl.BlockSpec((NUM_CLASS, EMBED), lambda i: (0, 0)),
    ],
    out_specs=pl.BlockSpec((PROJ_BLK, NUM_CLASS), lambda i: (i, 0)),
    out_shape=jax.ShapeDtypeStruct((VOCAB, NUM_CLASS), jnp.float32),
)


def _sc_bag_mean_body(text_hbm, p_hbm, bias_hbm, out_hbm,
                      idx_v, bufs, acc_v, bias_v, sems):
    wid = lax.axis_index("s") * NC + lax.axis_index("c")
    row0 = wid * NCHUNK
    pltpu.sync_copy(text_hbm.at[pl.ds(row0, NCHUNK)], idx_v)
    pltpu.sync_copy(bias_hbm, bias_v)
    bias = bias_v[...]

    def fire(g, b):
        pltpu.async_copy(p_hbm.at[idx_v.at[g]], bufs[b], sems[b])

    def wait(g, b):
        pltpu.make_async_copy(p_hbm.at[idx_v.at[g]], bufs[b], sems[b]).wait()

    def process(g, b):
        buf_v = bufs[b]
        for h in range(CHUNK_BAGS):
            acc = jnp.zeros((NUM_CLASS,), jnp.float32)
            for t in range(BAG):
                acc = acc + buf_v[h * BAG + t, :]
            acc_v[g * CHUNK_BAGS + h, :] = acc * (1.0 / BAG) + bias

    for b in range(NBUF):
        fire(b, b)

    def group(i, carry):
        for b in range(NBUF):
            g = i * NBUF + b
            wait(g, b)
            process(g, b)
            nxt = g + NBUF

            @pl.when(nxt < NCHUNK)
            def _():
                fire(nxt, b)

        return carry

    lax.fori_loop(0, NCHUNK // NBUF, group, 0)
    pltpu.sync_copy(acc_v, out_hbm.at[pl.ds(wid * BAGS_PER_W, BAGS_PER_W)])


_sc_bag_mean = pl.kernel(
    _sc_bag_mean_body,
    out_type=jax.ShapeDtypeStruct((BATCH, NUM_CLASS), jnp.float32),
    mesh=plsc.VectorSubcoreMesh(
        core_axis_name="c", subcore_axis_name="s",
        num_cores=NC, num_subcores=NS),
    scratch_types=[
        pltpu.VMEM((NCHUNK, CHUNK_TOK), jnp.int32),
        [pltpu.VMEM((CHUNK_TOK, NUM_CLASS), jnp.float32) for _ in range(NBUF)],
        pltpu.VMEM((BAGS_PER_W, NUM_CLASS), jnp.float32),
        pltpu.VMEM((NUM_CLASS,), jnp.float32),
        [pltpu.SemaphoreType.DMA for _ in range(NBUF)],
    ],
    compiler_params=pltpu.CompilerParams(use_tc_tiling_on_sc=False),
)


def kernel(text, offsets, emb_weight, fc_weight, fc_bias):
    del offsets
    proj = _tc_project(emb_weight.T, fc_weight)
    text2d = text.reshape(BATCH * BAG // CHUNK_TOK, CHUNK_TOK).astype(jnp.int32)
    return _sc_bag_mean(text2d, proj, fc_bias)


# speedup vs baseline: 62.7625x; 1.7469x over previous
import functools

import jax
import jax.numpy as jnp
from jax import lax
from jax.experimental import pallas as pl
from jax.experimental.pallas import tpu as pltpu
from jax.experimental.pallas import tpu_sc as plsc

VOCAB = 1000000
EMBED = 64
NUM_CLASS = 16
BATCH = 4096
BAG = 50

NC = 2
NS = 16
LANES = 16
NW = NC * NS

BAGS_PER_W = BATCH // NW
CHUNK_BAGS = 2
CHUNK_TOK = CHUNK_BAGS * BAG
NCHUNK = BAGS_PER_W // CHUNK_BAGS
NBUF = 4

PROJ_BLK = 16384
NPROJ = pl.cdiv(VOCAB, PROJ_BLK)
VOCAB_PAD = NPROJ * PROJ_BLK


PACK = 128 // NUM_CLASS


def _tc_project_body(et_ref, w_ref, o_ref):
    q = lax.dot_general(
        w_ref[...], et_ref[...], (((1,), (0,)), ((), ())),
        preferred_element_type=jnp.float32)
    q3 = q.T.reshape(PROJ_BLK // PACK, PACK, NUM_CLASS)
    for k in range(PACK):
        o_ref[:, k * NUM_CLASS:(k + 1) * NUM_CLASS] = q3[:, k, :]


_tc_project = pl.pallas_call(
    _tc_project_body,
    grid=(NPROJ,),
    in_specs=[
        pl.BlockSpec((EMBED, PROJ_BLK), lambda i: (0, i)),
        pl.BlockSpec((NUM_CLASS, EMBED), lambda i: (0, 0)),
    ],
    out_specs=pl.BlockSpec((PROJ_BLK // PACK, PACK * NUM_CLASS), lambda i: (i, 0)),
    out_shape=jax.ShapeDtypeStruct((VOCAB_PAD // PACK, PACK * NUM_CLASS), jnp.float32),
)


def _sc_bag_mean_body(text_hbm, p_hbm, bias_hbm, out_hbm,
                      idx_v, bufs, acc_v, bias_v, sems):
    wid = lax.axis_index("s") * NC + lax.axis_index("c")
    row0 = wid * NCHUNK
    pltpu.sync_copy(text_hbm.at[pl.ds(row0, NCHUNK)], idx_v)
    pltpu.sync_copy(bias_hbm, bias_v)
    bias = bias_v[...]

    def fire(g, b):
        pltpu.async_copy(p_hbm.at[idx_v.at[g]], bufs[b], sems[b])

    def wait(g, b):
        pltpu.make_async_copy(p_hbm.at[idx_v.at[g]], bufs[b], sems[b]).wait()

    def process(g, b):
        buf_v = bufs[b]
        for h in range(CHUNK_BAGS):
            acc = jnp.zeros((NUM_CLASS,), jnp.float32)
            for t in range(BAG):
                acc = acc + buf_v[h * BAG + t, :]
            acc_v[g * CHUNK_BAGS + h, :] = acc * (1.0 / BAG) + bias

    for b in range(NBUF):
        fire(b, b)

    def group(i, carry):
        for b in range(NBUF):
            g = i * NBUF + b
            wait(g, b)
            process(g, b)
            nxt = g + NBUF

            @pl.when(nxt < NCHUNK)
            def _():
                fire(nxt, b)

        return carry

    lax.fori_loop(0, NCHUNK // NBUF, group, 0)
    pltpu.sync_copy(acc_v, out_hbm.at[pl.ds(wid * BAGS_PER_W, BAGS_PER_W)])


_sc_bag_mean = pl.kernel(
    _sc_bag_mean_body,
    out_type=jax.ShapeDtypeStruct((BATCH, NUM_CLASS), jnp.float32),
    mesh=plsc.VectorSubcoreMesh(
        core_axis_name="c", subcore_axis_name="s",
        num_cores=NC, num_subcores=NS),
    scratch_types=[
        pltpu.VMEM((NCHUNK, CHUNK_TOK), jnp.int32),
        [pltpu.VMEM((CHUNK_TOK, NUM_CLASS), jnp.float32) for _ in range(NBUF)],
        pltpu.VMEM((BAGS_PER_W, NUM_CLASS), jnp.float32),
        pltpu.VMEM((NUM_CLASS,), jnp.float32),
        [pltpu.SemaphoreType.DMA for _ in range(NBUF)],
    ],
    compiler_params=pltpu.CompilerParams(use_tc_tiling_on_sc=False),
)


def kernel(text, offsets, emb_weight, fc_weight, fc_bias):
    del offsets
    proj = _tc_project(emb_weight.T, fc_weight)
    proj = proj.reshape(VOCAB_PAD, NUM_CLASS)
    text2d = text.reshape(BATCH * BAG // CHUNK_TOK, CHUNK_TOK).astype(jnp.int32)
    return _sc_bag_mean(text2d, proj, fc_bias)


# speedup vs baseline: 149.0342x; 2.3746x over previous
import jax
import jax.numpy as jnp
from jax import lax
from jax.experimental import pallas as pl
from jax.experimental.pallas import tpu as pltpu
from jax.experimental.pallas import tpu_sc as plsc

VOCAB = 1000000
EMBED = 64
NUM_CLASS = 16
BATCH = 4096
BAG = 50

NC = 2
NS = 16
LANES = 16
NW = NC * NS

BAGS_PER_W = BATCH // NW
TOK_PER_W = BAGS_PER_W * BAG
CHUNK_BAGS = 2
CHUNK_TOK = CHUNK_BAGS * BAG
NCHUNK = BAGS_PER_W // CHUNK_BAGS
NBUF = 4

PACK = 128 // NUM_CLASS
PROJ_R = 4096
VP8_LOG2 = 17
VP8 = 1 << VP8_LOG2
VOCAB_PAD = VP8 * PACK
NPROJ = VP8 // PROJ_R


def _tc_project_body(*refs):
    ets, w_ref, o_ref = refs[:PACK], refs[PACK], refs[PACK + 1]
    w = w_ref[...]
    qs = [
        lax.dot_general(w, ek[...], (((1,), (0,)), ((), ())),
                        preferred_element_type=jnp.float32)
        for ek in ets
    ]
    q_all = jnp.concatenate(qs, axis=0)
    o_ref[...] = q_all.T


MAX_ET_BLK = VOCAB // PROJ_R


def _et_spec(k):
    return pl.BlockSpec(
        (EMBED, PROJ_R),
        lambda i, k=k: (0, jnp.minimum(k * NPROJ + i, MAX_ET_BLK)))


_tc_project = pl.pallas_call(
    _tc_project_body,
    grid=(NPROJ,),
    in_specs=[_et_spec(k) for k in range(PACK)] + [
        pl.BlockSpec((NUM_CLASS, EMBED), lambda i: (0, 0)),
    ],
    out_specs=pl.BlockSpec((PROJ_R, PACK * NUM_CLASS), lambda i: (i, 0)),
    out_shape=jax.ShapeDtypeStruct((VP8, PACK * NUM_CLASS), jnp.float32),
)


def _tc_remap_body(x_ref, o_ref):
    v = x_ref[...]
    o_ref[...] = (v & (VP8 - 1)) * PACK + lax.shift_right_arithmetic(v, VP8_LOG2)


_tc_remap = pl.pallas_call(
    _tc_remap_body,
    out_shape=jax.ShapeDtypeStruct((BATCH * BAG // 128, 128), jnp.int32),
)


def _sc_bag_mean_body(text_hbm, p_hbm, bias_hbm, out_hbm,
                      idx_v, bufs, acc_v, bias_v, sems):
    wid = lax.axis_index("s") * NC + lax.axis_index("c")
    pltpu.sync_copy(text_hbm.at[pl.ds(wid * NCHUNK, NCHUNK)], idx_v)
    pltpu.sync_copy(bias_hbm, bias_v)
    bias = bias_v[...]

    def fire(g, b):
        pltpu.async_copy(p_hbm.at[idx_v.at[g]], bufs[b], sems[b])

    def wait(g, b):
        pltpu.make_async_copy(p_hbm.at[idx_v.at[g]], bufs[b], sems[b]).wait()

    def process(g, b):
        buf_v = bufs[b]
        for h in range(CHUNK_BAGS):
            acc = jnp.zeros((NUM_CLASS,), jnp.float32)
            for t in range(BAG):
                acc = acc + buf_v[h * BAG + t, :]
            acc_v[g * CHUNK_BAGS + h, :] = acc * (1.0 / BAG) + bias

    for b in range(NBUF):
        fire(b, b)

    def group(i, carry):
        for b in range(NBUF):
            g = i * NBUF + b
            wait(g, b)
            process(g, b)
            nxt = g + NBUF

            @pl.when(nxt < NCHUNK)
            def _():
                fire(nxt, b)

        return carry

    lax.fori_loop(0, NCHUNK // NBUF, group, 0)
    pltpu.sync_copy(acc_v, out_hbm.at[pl.ds(wid * BAGS_PER_W, BAGS_PER_W)])


_sc_bag_mean = pl.kernel(
    _sc_bag_mean_body,
    out_type=jax.ShapeDtypeStruct((BATCH, NUM_CLASS), jnp.float32),
    mesh=plsc.VectorSubcoreMesh(
        core_axis_name="c", subcore_axis_name="s",
        num_cores=NC, num_subcores=NS),
    scratch_types=[
        pltpu.VMEM((NCHUNK, CHUNK_TOK), jnp.int32),
        [pltpu.VMEM((CHUNK_TOK, NUM_CLASS), jnp.float32) for _ in range(NBUF)],
        pltpu.VMEM((BAGS_PER_W, NUM_CLASS), jnp.float32),
        pltpu.VMEM((NUM_CLASS,), jnp.float32),
        [pltpu.SemaphoreType.DMA for _ in range(NBUF)],
    ],
    compiler_params=pltpu.CompilerParams(use_tc_tiling_on_sc=False),
)


def kernel(text, offsets, emb_weight, fc_weight, fc_bias):
    del offsets
    et = emb_weight.T
    proj = _tc_project(*([et] * PACK), fc_weight)
    proj = proj.reshape(VOCAB_PAD, NUM_CLASS)
    idx = _tc_remap(text.astype(jnp.int32).reshape(BATCH * BAG // 128, 128))
    idx2d = idx.reshape(BATCH * BAG // CHUNK_TOK, CHUNK_TOK)
    return _sc_bag_mean(idx2d, proj, fc_bias)


# speedup vs baseline: 151.5673x; 1.0170x over previous
import jax
import jax.numpy as jnp
from jax import lax
from jax.experimental import pallas as pl
from jax.experimental.pallas import tpu as pltpu
from jax.experimental.pallas import tpu_sc as plsc

VOCAB = 1000000
EMBED = 64
NUM_CLASS = 16
BATCH = 4096
BAG = 50

NC = 2
NS = 16
LANES = 16
NW = NC * NS

BAGS_PER_W = BATCH // NW
TOK_PER_W = BAGS_PER_W * BAG
CHUNK_BAGS = 2
CHUNK_TOK = CHUNK_BAGS * BAG
NCHUNK = BAGS_PER_W // CHUNK_BAGS
NBUF = 4

PACK = 128 // NUM_CLASS
PROJ_R = 8192
VP8_LOG2 = 17
VP8 = 1 << VP8_LOG2
VOCAB_PAD = VP8 * PACK
NPROJ = VP8 // PROJ_R


TEXT_ROWS = BATCH * BAG // 128


def _tc_project_body(*refs):
    ets, w_ref, t_ref = refs[:PACK], refs[PACK], refs[PACK + 1]
    o_ref, oi_ref = refs[PACK + 2], refs[PACK + 3]
    w = w_ref[...]
    qs = [
        lax.dot_general(w, ek[...], (((1,), (0,)), ((), ())),
                        preferred_element_type=jnp.float32)
        for ek in ets
    ]
    q_all = jnp.concatenate(qs, axis=0)
    o_ref[...] = q_all.T
    v = t_ref[...]
    oi_ref[...] = (v & (VP8 - 1)) * PACK + lax.shift_right_arithmetic(v, VP8_LOG2)


MAX_ET_BLK = VOCAB // PROJ_R


def _et_spec(k):
    return pl.BlockSpec(
        (EMBED, PROJ_R),
        lambda i, k=k: (0, jnp.minimum(k * NPROJ + i, MAX_ET_BLK)))


_tc_project = pl.pallas_call(
    _tc_project_body,
    grid=(NPROJ,),
    in_specs=[_et_spec(k) for k in range(PACK)] + [
        pl.BlockSpec((NUM_CLASS, EMBED), lambda i: (0, 0)),
        pl.BlockSpec((TEXT_ROWS, 128), lambda i: (0, 0)),
    ],
    out_specs=[
        pl.BlockSpec((PROJ_R, PACK * NUM_CLASS), lambda i: (i, 0)),
        pl.BlockSpec((TEXT_ROWS, 128), lambda i: (0, 0)),
    ],
    out_shape=[
        jax.ShapeDtypeStruct((VP8, PACK * NUM_CLASS), jnp.float32),
        jax.ShapeDtypeStruct((TEXT_ROWS, 128), jnp.int32),
    ],
)


def _sc_bag_mean_body(text_hbm, p_hbm, bias_hbm, out_hbm,
                      idx_v, bufs, acc_v, bias_v, sems):
    wid = lax.axis_index("s") * NC + lax.axis_index("c")
    pltpu.sync_copy(text_hbm.at[pl.ds(wid * NCHUNK, NCHUNK)], idx_v)
    pltpu.sync_copy(bias_hbm, bias_v)
    bias = bias_v[...]

    def fire(g, b):
        pltpu.async_copy(p_hbm.at[idx_v.at[g]], bufs[b], sems[b])

    def wait(g, b):
        pltpu.make_async_copy(p_hbm.at[idx_v.at[g]], bufs[b], sems[b]).wait()

    def process(g, b):
        buf_v = bufs[b]
        for h in range(CHUNK_BAGS):
            acc = jnp.zeros((NUM_CLASS,), jnp.float32)
            for t in range(BAG):
                acc = acc + buf_v[h * BAG + t, :]
            acc_v[g * CHUNK_BAGS + h, :] = acc * (1.0 / BAG) + bias

    for b in range(NBUF):
        fire(b, b)

    def group(i, carry):
        for b in range(NBUF):
            g = i * NBUF + b
            wait(g, b)
            process(g, b)
            nxt = g + NBUF

            @pl.when(nxt < NCHUNK)
            def _():
                fire(nxt, b)

        return carry

    lax.fori_loop(0, NCHUNK // NBUF, group, 0)
    pltpu.sync_copy(acc_v, out_hbm.at[pl.ds(wid * BAGS_PER_W, BAGS_PER_W)])


_sc_bag_mean = pl.kernel(
    _sc_bag_mean_body,
    out_type=jax.ShapeDtypeStruct((BATCH, NUM_CLASS), jnp.float32),
    mesh=plsc.VectorSubcoreMesh(
        core_axis_name="c", subcore_axis_name="s",
        num_cores=NC, num_subcores=NS),
    scratch_types=[
        pltpu.VMEM((NCHUNK, CHUNK_TOK), jnp.int32),
        [pltpu.VMEM((CHUNK_TOK, NUM_CLASS), jnp.float32) for _ in range(NBUF)],
        pltpu.VMEM((BAGS_PER_W, NUM_CLASS), jnp.float32),
        pltpu.VMEM((NUM_CLASS,), jnp.float32),
        [pltpu.SemaphoreType.DMA for _ in range(NBUF)],
    ],
    compiler_params=pltpu.CompilerParams(use_tc_tiling_on_sc=False),
)


def kernel(text, offsets, emb_weight, fc_weight, fc_bias):
    del offsets
    et = emb_weight.T
    text128 = text.astype(jnp.int32).reshape(TEXT_ROWS, 128)
    proj, idx = _tc_project(*([et] * PACK), fc_weight, text128)
    proj = proj.reshape(VOCAB_PAD, NUM_CLASS)
    idx2d = idx.reshape(BATCH * BAG // CHUNK_TOK, CHUNK_TOK)
    return _sc_bag_mean(idx2d, proj, fc_bias)
